# baseline (speedup 1.0000x reference)
import functools

import jax
import jax.numpy as jnp
from jax import lax
from jax.experimental import pallas as pl
from jax.experimental.pallas import tpu as pltpu
from jax.experimental.pallas import tpu_sc as plsc

_D = 1280
_VPAD = 72
_RPS = 8
_NBUF = 2


def _build_lookup(batch: int, seqlen: int, chunk: int):
    info = plsc.get_sparse_core_info()
    nc, ns = info.num_cores, info.num_subcores
    num_tokens = batch * seqlen
    tpw = num_tokens // (nc * ns)
    assert num_tokens % (nc * ns) == 0 and tpw % chunk == 0 and seqlen % tpw == 0
    nchunks = tpw // chunk
    wpb = seqlen // tpw
    nbuilders = _VPAD // _RPS

    mesh = plsc.VectorSubcoreMesh(core_axis_name="c", subcore_axis_name="s")

    @functools.partial(
        pl.kernel,
        mesh=mesh,
        out_type=jax.ShapeDtypeStruct((batch, seqlen, 2 * _D), jnp.float32),
        scratch_types=[
            pltpu.VMEM((tpw,), jnp.int32),
            pltpu.VMEM((16,), jnp.int32),
            pltpu.VMEM((_RPS, _D), jnp.float32),
            pltpu.VMEM((_NBUF, chunk, 2 * _D), jnp.float32),
            pltpu.VMEM_SHARED((_VPAD, 2 * _D), jnp.float32),
            pltpu.SemaphoreType.DMA,
            pltpu.SemaphoreType.DMA,
        ],
    )
    def lookup(x_hbm, c2a_hbm, codon_hbm, aa_hbm, out_hbm,
               x_v, idx_v, stage_v, buf, fused_sp, gsem, wsem):
        sc = lax.axis_index("c")
        sid = lax.axis_index("s")
        wid = sid * nc + sc
        b = wid // wpb
        l0 = (wid % wpb) * tpw

        pltpu.sync_copy(x_hbm.at[b, pl.ds(l0, tpw)], x_v)

        t0 = (nbuilders - 1) * _RPS
        vtail = codon_hbm.shape[0] - t0

        @pl.when(sid < nbuilders - 1)
        def _build():
            r0 = sid * _RPS
            pltpu.sync_copy(c2a_hbm.at[pl.ds(r0, _RPS)], idx_v.at[pl.ds(0, _RPS)])
            ga = pltpu.async_copy(
                aa_hbm.at[idx_v.at[pl.ds(0, _RPS)]], stage_v, gsem)
            pltpu.sync_copy(codon_hbm.at[pl.ds(r0, _RPS)],
                            fused_sp.at[pl.ds(r0, _RPS), pl.ds(0, _D)])
            ga.wait()
            pltpu.sync_copy(stage_v,
                            fused_sp.at[pl.ds(r0, _RPS), pl.ds(_D, _D)])

        @pl.when(sid == nbuilders - 1)
        def _build_tail():
            idx_v[...] = jnp.zeros((16,), jnp.int32)
            pltpu.sync_copy(c2a_hbm.at[pl.ds(t0, vtail)],
                            idx_v.at[pl.ds(0, vtail)])
            ga = pltpu.async_copy(
                aa_hbm.at[idx_v.at[pl.ds(0, _RPS)]], stage_v, gsem)
            pltpu.sync_copy(codon_hbm.at[pl.ds(t0, vtail)],
                            fused_sp.at[pl.ds(t0, vtail), pl.ds(0, _D)])
            ga.wait()
            pltpu.sync_copy(stage_v,
                            fused_sp.at[pl.ds(t0, _RPS), pl.ds(_D, _D)])
        plsc.subcore_barrier()

        def wait_one_row():
            pltpu.make_async_copy(
                fused_sp.at[pl.ds(0, 1)],
                out_hbm.at[b, pl.ds(l0, 1)], wsem).wait()

        def body(c, _):
            xv = x_v[pl.ds(c * chunk, chunk)]
            for j in range(chunk):
                pltpu.async_copy(
                    fused_sp.at[pl.ds(xv[j], 1)],
                    out_hbm.at[b, pl.ds(l0 + c * chunk + j, 1)], wsem)

            @pl.when(c > 0)
            def _wait():
                for _ in range(chunk):
                    wait_one_row()
            return 0

        lax.fori_loop(0, nchunks, body, 0)
        for _ in range(chunk):
            wait_one_row()

    return lookup


def kernel(x, aa_table, codon_table, codon_to_aa):
    b, l = x.shape
    v = codon_table.shape[0]
    xi = x.astype(jnp.int32)
    c2a = codon_to_aa.astype(jnp.int32)
    lookup = _build_lookup(b, l, 16)
    return lookup(xi, c2a, codon_table, aa_table)


# speedup vs baseline: 1.0767x; 1.0767x over previous
import functools

import jax
import jax.numpy as jnp
from jax import lax
from jax.experimental import pallas as pl
from jax.experimental.pallas import tpu as pltpu
from jax.experimental.pallas import tpu_sc as plsc

_D = 1280
_VPAD = 72
_RPS = 8
_NBUF = 2


def _build_lookup(batch: int, seqlen: int, chunk: int):
    info = plsc.get_sparse_core_info()
    nc, ns = info.num_cores, info.num_subcores
    num_tokens = batch * seqlen
    tpw = num_tokens // (nc * ns)
    assert num_tokens % (nc * ns) == 0 and tpw % chunk == 0 and seqlen % tpw == 0
    nchunks = tpw // chunk
    wpb = seqlen // tpw
    nbuilders = _VPAD // _RPS

    mesh = plsc.VectorSubcoreMesh(core_axis_name="c", subcore_axis_name="s")

    @functools.partial(
        pl.kernel,
        mesh=mesh,
        out_type=jax.ShapeDtypeStruct((batch, seqlen, 2 * _D), jnp.float32),
        scratch_types=[
            pltpu.VMEM((tpw,), jnp.int32),
            pltpu.VMEM((16,), jnp.int32),
            pltpu.VMEM((_RPS, _D), jnp.float32),
            pltpu.VMEM((_NBUF, chunk, 2 * _D), jnp.float32),
            pltpu.VMEM_SHARED((_VPAD, 2 * _D), jnp.float32),
            pltpu.SemaphoreType.DMA,
            pltpu.SemaphoreType.DMA,
        ],
    )
    def lookup(x_hbm, c2a_hbm, codon_hbm, aa_hbm, out_hbm,
               x_v, idx_v, stage_v, buf, fused_sp, gsem, wsem):
        sc = lax.axis_index("c")
        sid = lax.axis_index("s")
        wid = sid * nc + sc
        b = wid // wpb
        l0 = (wid % wpb) * tpw

        pltpu.sync_copy(x_hbm.at[b, pl.ds(l0, tpw)], x_v)

        t0 = (nbuilders - 1) * _RPS
        vtail = codon_hbm.shape[0] - t0

        @pl.when(sid < nbuilders - 1)
        def _build():
            r0 = sid * _RPS
            pltpu.sync_copy(c2a_hbm.at[pl.ds(r0, _RPS)], idx_v.at[pl.ds(0, _RPS)])
            ga = pltpu.async_copy(
                aa_hbm.at[idx_v.at[pl.ds(0, _RPS)]], stage_v, gsem)
            pltpu.sync_copy(codon_hbm.at[pl.ds(r0, _RPS)],
                            fused_sp.at[pl.ds(r0, _RPS), pl.ds(0, _D)])
            ga.wait()
            pltpu.sync_copy(stage_v,
                            fused_sp.at[pl.ds(r0, _RPS), pl.ds(_D, _D)])

        @pl.when(sid == nbuilders - 1)
        def _build_tail():
            idx_v[...] = jnp.zeros((16,), jnp.int32)
            pltpu.sync_copy(c2a_hbm.at[pl.ds(t0, vtail)],
                            idx_v.at[pl.ds(0, vtail)])
            ga = pltpu.async_copy(
                aa_hbm.at[idx_v.at[pl.ds(0, _RPS)]], stage_v, gsem)
            pltpu.sync_copy(codon_hbm.at[pl.ds(t0, vtail)],
                            fused_sp.at[pl.ds(t0, vtail), pl.ds(0, _D)])
            ga.wait()
            pltpu.sync_copy(stage_v,
                            fused_sp.at[pl.ds(t0, _RPS), pl.ds(_D, _D)])
        plsc.subcore_barrier()

        def wait_one_write():
            pltpu.make_async_copy(
                buf.at[0], out_hbm.at[b, pl.ds(l0, chunk)], wsem).wait()

        def body(c, _):
            slot = lax.rem(c, _NBUF)

            @pl.when(c >= _NBUF)
            def _wait():
                wait_one_write()

            xv = x_v[pl.ds(c * chunk, chunk)]
            rows = []
            for j in range(chunk):
                rows.append(pltpu.async_copy(
                    fused_sp.at[pl.ds(xv[j], 1)],
                    buf.at[slot, pl.ds(j, 1)], gsem))
            for r in rows:
                r.wait()
            pltpu.async_copy(
                buf.at[slot],
                out_hbm.at[b, pl.ds(l0 + c * chunk, chunk)], wsem)
            return 0

        lax.fori_loop(0, nchunks, body, 0)
        for _ in range(_NBUF):
            wait_one_write()

    return lookup


def kernel(x, aa_table, codon_table, codon_to_aa):
    b, l = x.shape
    v = codon_table.shape[0]
    xi = x.astype(jnp.int32)
    c2a = codon_to_aa.astype(jnp.int32)
    lookup = _build_lookup(b, l, 16)
    return lookup(xi, c2a, codon_table, aa_table)
